# baseline (speedup 1.0000x reference)
import jax
import jax.numpy as jnp
from jax import lax
from jax.experimental import pallas as pl
from jax.experimental.pallas import tpu as pltpu
from jax.experimental.pallas import tpu_sc as plsc

EMB = 64
WINDOW = 200
PATH = 20
NC = 2
NS = 16
L = 16
LANES = 128
WPT = 8
NTILES_W = WINDOW // WPT
PFULL = PATH // WPT
PREM = PATH - PFULL * WPT


def _sc_body(ut_hbm, vt_hbm, words_hbm, path_hbm, partials_hbm, theta_hbm,
             idx_v, buf_v, acc_v, sem):
    wid = lax.axis_index("s") * NC + lax.axis_index("c")
    rowi = lax.iota(jnp.int32, L)

    def split_idx():
        ivec = idx_v[...]
        return (ivec // LANES) * LANES, ivec % LANES

    def fire(tab, tcol, n):
        def body(r, carry):
            rvec = jnp.broadcast_to(r, (L,))
            cb = pl.multiple_of(tcol.at[rvec].get(mode="promise_in_bounds")[0],
                                LANES)
            pltpu.async_copy(tab.at[:, pl.ds(cb, LANES)], buf_v.at[r], sem)
            return carry
        lax.fori_loop(0, n, body, 0)

    def extract(lane, r):
        rvec = jnp.broadcast_to(r, (L,))
        laneb = lane.at[rvec].get(mode="promise_in_bounds")
        return [plsc.load_gather(buf_v, [rvec, rowi + (c * L), laneb])
                for c in range(EMB // L)]

    def theta_rows(pbase, n):
        tcol, lane = split_idx()
        fire(vt_hbm, tcol, n)

        def drain(r, carry):
            pltpu.make_async_copy(vt_hbm.at[:, pl.ds(0, LANES)], buf_v.at[r],
                                  sem).wait()
            g = extract(lane, r)
            for c in range(EMB // L):
                acc_v[pl.ds(c * L, L)] = g[c]
            pltpu.sync_copy(acc_v, theta_hbm.at[pbase + r])
            return carry

        lax.fori_loop(0, n, drain, 0)

    @pl.when(wid < NTILES_W)
    def _():
        base = pl.multiple_of(wid * WPT, WPT)
        pltpu.sync_copy(words_hbm.at[pl.ds(base, WPT)], idx_v.at[pl.ds(0, WPT)])
        tcol, lane = split_idx()
        fire(ut_hbm, tcol, WPT)

        def drain(r, accs):
            pltpu.make_async_copy(ut_hbm.at[:, pl.ds(0, LANES)], buf_v.at[r],
                                  sem).wait()
            g = extract(lane, r)
            return tuple(a + b for a, b in zip(accs, g))

        accs = lax.fori_loop(0, WPT, drain,
                             tuple(jnp.zeros((L,), jnp.float32)
                                   for _ in range(EMB // L)))
        for c in range(EMB // L):
            acc_v[pl.ds(c * L, L)] = accs[c]
        pltpu.sync_copy(acc_v, partials_hbm.at[wid])

    @pl.when(jnp.logical_and(wid >= NTILES_W, wid < NTILES_W + PFULL))
    def _():
        pbase = pl.multiple_of((wid - NTILES_W) * WPT, WPT)
        pltpu.sync_copy(path_hbm.at[pl.ds(pbase, WPT)], idx_v.at[pl.ds(0, WPT)])
        theta_rows(pbase, WPT)

    @pl.when(wid == NTILES_W + PFULL)
    def _():
        pltpu.sync_copy(path_hbm.at[pl.ds(PFULL * WPT, PREM)],
                        idx_v.at[pl.ds(0, PREM)])
        theta_rows(PFULL * WPT, PREM)


def _sc_gather(words, h_path, ut, vt):
    mesh = plsc.VectorSubcoreMesh(core_axis_name="c", subcore_axis_name="s")
    f = pl.kernel(
        _sc_body,
        out_type=(
            jax.ShapeDtypeStruct((NTILES_W, EMB), jnp.float32),
            jax.ShapeDtypeStruct((PATH, EMB), jnp.float32),
        ),
        mesh=mesh,
        scratch_types=[
            pltpu.VMEM((L,), jnp.int32),
            pltpu.VMEM((WPT, EMB, LANES), jnp.float32),
            pltpu.VMEM((EMB,), jnp.float32),
            pltpu.SemaphoreType.DMA,
        ],
        compiler_params=pltpu.CompilerParams(needs_layout_passes=False),
    )
    return f(ut, vt, words, h_path)


def _tc_body(partials_ref, theta_ref, hcode_ref, out_ref):
    xw = jnp.sum(partials_ref[...], axis=0, keepdims=True)
    t = jnp.sum(theta_ref[...] * xw, axis=1)
    z = jax.nn.sigmoid(t)
    hc = hcode_ref[...]
    loss = jnp.log(z) * hc + jnp.log(1.0 - z) * (1.0 - hc)
    out_ref[...] = jnp.sum(loss).reshape(1, 1)


def _tc_finish(partials, theta, h_code):
    return pl.pallas_call(
        _tc_body,
        out_shape=jax.ShapeDtypeStruct((1, 1), jnp.float32),
    )(partials, theta, h_code)


def kernel(words, h_code, h_path, u_emb, v_emb):
    words = words.astype(jnp.int32)
    h_path = h_path.astype(jnp.int32)
    partials, theta = _sc_gather(words, h_path, u_emb.T, v_emb.T)
    return _tc_finish(partials, theta, h_code)


# speedup vs baseline: 1.0002x; 1.0002x over previous
import jax
import jax.numpy as jnp
from jax import lax
from jax.experimental import pallas as pl
from jax.experimental.pallas import tpu as pltpu
from jax.experimental.pallas import tpu_sc as plsc

EMB = 64
WINDOW = 200
PATH = 20
NC = 2
NS = 16
L = 16
LANES = 128
WPT = 8
NTILES_W = WINDOW // WPT
PFULL = PATH // WPT
PREM = PATH - PFULL * WPT


def _sc_body(ut_hbm, vt_hbm, words_hbm, path_hbm, partials_hbm, theta_hbm,
             idx_v, buf_v, acc_v, sem):
    wid = lax.axis_index("s") * NC + lax.axis_index("c")
    rowi = lax.iota(jnp.int32, L)

    def split_idx():
        ivec = idx_v[...]
        return (ivec // LANES) * LANES, ivec % LANES

    def fire(tab, tcol, n):
        def body(r, carry):
            rvec = jnp.broadcast_to(r, (L,))
            cb = pl.multiple_of(tcol.at[rvec].get(mode="promise_in_bounds")[0],
                                LANES)
            pltpu.async_copy(tab.at[:, pl.ds(cb, LANES)], buf_v.at[r], sem)
            return carry
        lax.fori_loop(0, n, body, 0)

    def extract(lane, r):
        rvec = jnp.broadcast_to(r, (L,))
        laneb = lane.at[rvec].get(mode="promise_in_bounds")
        return [plsc.load_gather(buf_v, [rvec, rowi + (c * L), laneb])
                for c in range(EMB // L)]

    def theta_rows(pbase, n):
        tcol, lane = split_idx()
        fire(vt_hbm, tcol, n)

        def drain(r, carry):
            pltpu.make_async_copy(vt_hbm.at[:, pl.ds(0, LANES)], buf_v.at[r],
                                  sem).wait()
            g = extract(lane, r)
            for c in range(EMB // L):
                acc_v[pl.ds(c * L, L)] = g[c]
            pltpu.sync_copy(acc_v, theta_hbm.at[pbase + r])
            return carry

        lax.fori_loop(0, n, drain, 0)

    @pl.when(wid < NTILES_W)
    def _():
        base = pl.multiple_of(wid * WPT, WPT)
        pltpu.sync_copy(words_hbm.at[pl.ds(base, WPT)], idx_v.at[pl.ds(0, WPT)])
        tcol, lane = split_idx()
        fire(ut_hbm, tcol, WPT)

        def drain(r, accs):
            pltpu.make_async_copy(ut_hbm.at[:, pl.ds(0, LANES)], buf_v.at[r],
                                  sem).wait()
            g = extract(lane, r)
            return tuple(a + b for a, b in zip(accs, g))

        accs = lax.fori_loop(0, WPT, drain,
                             tuple(jnp.zeros((L,), jnp.float32)
                                   for _ in range(EMB // L)))
        for c in range(EMB // L):
            acc_v[pl.ds(c * L, L)] = accs[c]
        pltpu.sync_copy(acc_v, partials_hbm.at[wid])

    @pl.when(jnp.logical_and(wid >= NTILES_W, wid < NTILES_W + PFULL))
    def _():
        pbase = pl.multiple_of((wid - NTILES_W) * WPT, WPT)
        pltpu.sync_copy(path_hbm.at[pl.ds(pbase, WPT)], idx_v.at[pl.ds(0, WPT)])
        theta_rows(pbase, WPT)

    @pl.when(wid == NTILES_W + PFULL)
    def _():
        pltpu.sync_copy(path_hbm.at[pl.ds(PFULL * WPT, PREM)],
                        idx_v.at[pl.ds(0, PREM)])
        theta_rows(PFULL * WPT, PREM)


def _sc_gather(words, h_path, ut, vt):
    mesh = plsc.VectorSubcoreMesh(core_axis_name="c", subcore_axis_name="s")
    f = pl.kernel(
        _sc_body,
        out_type=(
            jax.ShapeDtypeStruct((NTILES_W, EMB), jnp.float32),
            jax.ShapeDtypeStruct((PATH, EMB), jnp.float32),
        ),
        mesh=mesh,
        scratch_types=[
            pltpu.VMEM((L,), jnp.int32),
            pltpu.VMEM((WPT, EMB, LANES), jnp.float32),
            pltpu.VMEM((EMB,), jnp.float32),
            pltpu.SemaphoreType.DMA,
        ],
        compiler_params=pltpu.CompilerParams(needs_layout_passes=False, skip_device_barrier=True),
    )
    return f(ut, vt, words, h_path)


def _tc_body(partials_ref, theta_ref, hcode_ref, out_ref):
    xw = jnp.sum(partials_ref[...], axis=0, keepdims=True)
    t = jnp.sum(theta_ref[...] * xw, axis=1)
    z = jax.nn.sigmoid(t)
    hc = hcode_ref[...]
    loss = jnp.log(z) * hc + jnp.log(1.0 - z) * (1.0 - hc)
    out_ref[...] = jnp.sum(loss).reshape(1, 1)


def _tc_finish(partials, theta, h_code):
    return pl.pallas_call(
        _tc_body,
        out_shape=jax.ShapeDtypeStruct((1, 1), jnp.float32),
        compiler_params=pltpu.CompilerParams(skip_device_barrier=True),
    )(partials, theta, h_code)


def kernel(words, h_code, h_path, u_emb, v_emb):
    words = words.astype(jnp.int32)
    h_path = h_path.astype(jnp.int32)
    partials, theta = _sc_gather(words, h_path, u_emb.T, v_emb.T)
    return _tc_finish(partials, theta, h_code)
